# baseline (speedup 1.0000x reference)
import dataclasses
import functools

import jax
import jax.numpy as jnp
from jax import lax
from jax.experimental import pallas as pl
from jax.experimental.pallas import tpu as pltpu
from jax.experimental.pallas import tpu_sc as plsc

N_NODES = 10000
N_EDGES = 320000
D_FEAT = 128
L = 16
NUM_CORES = 2
NUM_SUBCORES = 16
NW = NUM_CORES * NUM_SUBCORES
E_PER_TILE = N_EDGES // NW
W = 80
NWIN = E_PER_TILE // W


def _sqrt_vec(x):
    bits = plsc.bitcast(x, jnp.int32)
    y = plsc.bitcast(jnp.int32(0x5F3759DF) - (bits >> 1), jnp.float32)
    half = x * 0.5
    for _ in range(3):
        y = y * (1.5 - half * y * y)
    return x * y


def kernel(z, edge_index):
    src = edge_index[0]
    dst = edge_index[1]
    mesh = plsc.VectorSubcoreMesh(core_axis_name="c", subcore_axis_name="s")
    cp = pltpu.CompilerParams()
    if "needs_layout_passes" in pltpu.CompilerParams.__dataclass_fields__:
        cp = dataclasses.replace(cp, needs_layout_passes=False)

    @functools.partial(
        pl.kernel,
        out_type=jax.ShapeDtypeStruct((N_EDGES,), jnp.float32),
        mesh=mesh,
        compiler_params=cp,
        scratch_types=[
            pltpu.VMEM((W,), jnp.int32), pltpu.VMEM((W,), jnp.int32),
            pltpu.VMEM((W,), jnp.int32), pltpu.VMEM((W,), jnp.int32),
            pltpu.VMEM((W, D_FEAT), jnp.float32),
            pltpu.VMEM((W, D_FEAT), jnp.float32),
            pltpu.VMEM((W, D_FEAT), jnp.float32),
            pltpu.VMEM((W, D_FEAT), jnp.float32),
            pltpu.VMEM((W,), jnp.float32),
            pltpu.SemaphoreType.DMA, pltpu.SemaphoreType.DMA,
            pltpu.SemaphoreType.DMA, pltpu.SemaphoreType.DMA,
        ],
    )
    def sc_kernel(z_hbm, src_hbm, dst_hbm, out_hbm,
                  idx_s0, idx_d0, idx_s1, idx_d1,
                  rows_s0, rows_d0, rows_s1, rows_d1,
                  out_v, sem_s0, sem_d0, sem_s1, sem_d1):
        wid = lax.axis_index("s") * NUM_CORES + lax.axis_index("c")
        tile_base = wid * E_PER_TILE
        bufs = [
            (idx_s0, idx_d0, rows_s0, rows_d0, sem_s0, sem_d0),
            (idx_s1, idx_d1, rows_s1, rows_d1, sem_s1, sem_d1),
        ]
        lane = lax.broadcasted_iota(jnp.int32, (L,), 0)

        def stage(w, b):
            idx_s, idx_d, rows_s, rows_d, sem_s, sem_d = bufs[b]
            base = pl.multiple_of(tile_base + w * W, W)
            pltpu.sync_copy(src_hbm.at[pl.ds(base, W)], idx_s)
            pltpu.sync_copy(dst_hbm.at[pl.ds(base, W)], idx_d)
            pltpu.async_copy(z_hbm.at[idx_s], rows_s, sem_s)
            pltpu.async_copy(z_hbm.at[idx_d], rows_d, sem_d)

        def wait(b):
            idx_s, idx_d, rows_s, rows_d, sem_s, sem_d = bufs[b]
            pltpu.make_async_copy(z_hbm.at[idx_s], rows_s, sem_s).wait()
            pltpu.make_async_copy(z_hbm.at[idx_d], rows_d, sem_d).wait()

        def compute(w, b):
            _, _, rows_s, rows_d, _, _ = bufs[b]
            base = pl.multiple_of(tile_base + w * W, W)

            for g in range(W // L):
                e0 = g * L
                d2 = jnp.zeros((L,), jnp.float32)
                for k in range(L):
                    e = e0 + k
                    acc = jnp.zeros((L,), jnp.float32)
                    for j in range(D_FEAT // L):
                        vs = rows_s[e, pl.ds(j * L, L)]
                        vd = rows_d[e, pl.ds(j * L, L)]
                        df = vs - vd
                        acc = acc + df * df
                    d2 = jnp.where(lane == k, jnp.sum(acc), d2)
                out_v[pl.ds(e0, L)] = jnp.exp(-_sqrt_vec(d2))

            pltpu.sync_copy(out_v, out_hbm.at[pl.ds(base, W)])

        stage(0, 0)

        @pl.loop(0, (NWIN - 1) // 2)
        def _pair(p):
            w = p * 2
            stage(w + 1, 1)
            wait(0)
            compute(w, 0)
            stage(w + 2, 0)
            wait(1)
            compute(w + 1, 1)

        wait(0)
        compute(NWIN - 1, 0)

    return sc_kernel(z, src, dst)


# speedup vs baseline: 2.1475x; 2.1475x over previous
import dataclasses
import functools

import jax
import jax.numpy as jnp
from jax import lax
from jax.experimental import pallas as pl
from jax.experimental.pallas import tpu as pltpu
from jax.experimental.pallas import tpu_sc as plsc

N_NODES = 10000
N_EDGES = 320000
D_FEAT = 128
L = 16
NUM_CORES = 2
NUM_SUBCORES = 16
NW = NUM_CORES * NUM_SUBCORES
E_PER_TILE = N_EDGES // NW
W = 80
NWIN = E_PER_TILE // W


def _sqrt_vec(x):
    bits = plsc.bitcast(x, jnp.int32)
    y = plsc.bitcast(jnp.int32(0x5F3759DF) - (bits >> 1), jnp.float32)
    half = x * 0.5
    for _ in range(3):
        y = y * (1.5 - half * y * y)
    return x * y


def kernel(z, edge_index):
    src = edge_index[0]
    dst = edge_index[1]
    mesh = plsc.VectorSubcoreMesh(core_axis_name="c", subcore_axis_name="s")
    cp = pltpu.CompilerParams()
    if "needs_layout_passes" in pltpu.CompilerParams.__dataclass_fields__:
        cp = dataclasses.replace(cp, needs_layout_passes=False)

    @functools.partial(
        pl.kernel,
        out_type=jax.ShapeDtypeStruct((N_EDGES,), jnp.float32),
        mesh=mesh,
        compiler_params=cp,
        scratch_types=[
            pltpu.VMEM((W,), jnp.int32), pltpu.VMEM((W,), jnp.int32),
            pltpu.VMEM((W,), jnp.int32), pltpu.VMEM((W,), jnp.int32),
            pltpu.VMEM((W, D_FEAT), jnp.float32),
            pltpu.VMEM((W, D_FEAT), jnp.float32),
            pltpu.VMEM((W, D_FEAT), jnp.float32),
            pltpu.VMEM((W, D_FEAT), jnp.float32),
            pltpu.VMEM((W,), jnp.float32),
            pltpu.SemaphoreType.DMA, pltpu.SemaphoreType.DMA,
            pltpu.SemaphoreType.DMA, pltpu.SemaphoreType.DMA,
        ],
    )
    def sc_kernel(z_hbm, src_hbm, dst_hbm, out_hbm,
                  idx_s0, idx_d0, idx_s1, idx_d1,
                  rows_s0, rows_d0, rows_s1, rows_d1,
                  out_v, sem_s0, sem_d0, sem_s1, sem_d1):
        wid = lax.axis_index("s") * NUM_CORES + lax.axis_index("c")
        tile_base = wid * E_PER_TILE
        bufs = [
            (idx_s0, idx_d0, rows_s0, rows_d0, sem_s0, sem_d0),
            (idx_s1, idx_d1, rows_s1, rows_d1, sem_s1, sem_d1),
        ]
        lane = lax.broadcasted_iota(jnp.int32, (L,), 0)

        def stage(w, b):
            idx_s, idx_d, rows_s, rows_d, sem_s, sem_d = bufs[b]
            base = pl.multiple_of(tile_base + w * W, W)
            pltpu.sync_copy(src_hbm.at[pl.ds(base, W)], idx_s)
            pltpu.sync_copy(dst_hbm.at[pl.ds(base, W)], idx_d)
            pltpu.async_copy(z_hbm.at[idx_s], rows_s, sem_s)
            pltpu.async_copy(z_hbm.at[idx_d], rows_d, sem_d)

        def wait(b):
            idx_s, idx_d, rows_s, rows_d, sem_s, sem_d = bufs[b]
            pltpu.make_async_copy(z_hbm.at[idx_s], rows_s, sem_s).wait()
            pltpu.make_async_copy(z_hbm.at[idx_d], rows_d, sem_d).wait()

        def compute(w, b):
            _, _, rows_s, rows_d, _, _ = bufs[b]
            base = pl.multiple_of(tile_base + w * W, W)

            @pl.loop(0, W // L)
            def _group(g):
                e0 = pl.multiple_of(g * L, L)

                def body(k, res):
                    e = e0 + k
                    acc = jnp.zeros((L,), jnp.float32)
                    for j in range(D_FEAT // L):
                        vs = rows_s[e, pl.ds(j * L, L)]
                        vd = rows_d[e, pl.ds(j * L, L)]
                        df = vs - vd
                        acc = acc + df * df
                    return jnp.where(lane == k, jnp.sum(acc), res)

                d2 = lax.fori_loop(0, L, body, jnp.zeros((L,), jnp.float32),
                                   unroll=8)
                out_v[pl.ds(e0, L)] = jnp.exp(-_sqrt_vec(d2))

            pltpu.sync_copy(out_v, out_hbm.at[pl.ds(base, W)])

        stage(0, 0)

        @pl.loop(0, (NWIN - 1) // 2)
        def _pair(p):
            w = p * 2
            stage(w + 1, 1)
            wait(0)
            compute(w, 0)
            stage(w + 2, 0)
            wait(1)
            compute(w + 1, 1)

        wait(0)
        compute(NWIN - 1, 0)

    return sc_kernel(z, src, dst)


# speedup vs baseline: 2.2351x; 1.0408x over previous
import dataclasses
import functools

import jax
import jax.numpy as jnp
from jax import lax
from jax.experimental import pallas as pl
from jax.experimental.pallas import tpu as pltpu
from jax.experimental.pallas import tpu_sc as plsc

N_NODES = 10000
N_EDGES = 320000
D_FEAT = 128
L = 16
NUM_CORES = 2
NUM_SUBCORES = 16
NW = NUM_CORES * NUM_SUBCORES
E_PER_TILE = N_EDGES // NW
W = 80
NWIN = E_PER_TILE // W


def _sqrt_vec(x):
    bits = plsc.bitcast(x, jnp.int32)
    y = plsc.bitcast(jnp.int32(0x5F3759DF) - (bits >> 1), jnp.float32)
    half = x * 0.5
    for _ in range(3):
        y = y * (1.5 - half * y * y)
    return x * y


def kernel(z, edge_index):
    src = edge_index[0]
    dst = edge_index[1]
    mesh = plsc.VectorSubcoreMesh(core_axis_name="c", subcore_axis_name="s")
    cp = pltpu.CompilerParams()
    if "needs_layout_passes" in pltpu.CompilerParams.__dataclass_fields__:
        cp = dataclasses.replace(cp, needs_layout_passes=False)

    @functools.partial(
        pl.kernel,
        out_type=jax.ShapeDtypeStruct((N_EDGES,), jnp.float32),
        mesh=mesh,
        compiler_params=cp,
        scratch_types=[
            pltpu.VMEM((W,), jnp.int32), pltpu.VMEM((W,), jnp.int32),
            pltpu.VMEM((W,), jnp.int32), pltpu.VMEM((W,), jnp.int32),
            pltpu.VMEM((W, D_FEAT), jnp.float32),
            pltpu.VMEM((W, D_FEAT), jnp.float32),
            pltpu.VMEM((W, D_FEAT), jnp.float32),
            pltpu.VMEM((W, D_FEAT), jnp.float32),
            pltpu.VMEM((W,), jnp.float32),
            pltpu.SemaphoreType.DMA, pltpu.SemaphoreType.DMA,
            pltpu.SemaphoreType.DMA, pltpu.SemaphoreType.DMA,
        ],
    )
    def sc_kernel(z_hbm, src_hbm, dst_hbm, out_hbm,
                  idx_s0, idx_d0, idx_s1, idx_d1,
                  rows_s0, rows_d0, rows_s1, rows_d1,
                  out_v, sem_s0, sem_d0, sem_s1, sem_d1):
        wid = lax.axis_index("s") * NUM_CORES + lax.axis_index("c")
        tile_base = wid * E_PER_TILE
        bufs = [
            (idx_s0, idx_d0, rows_s0, rows_d0, sem_s0, sem_d0),
            (idx_s1, idx_d1, rows_s1, rows_d1, sem_s1, sem_d1),
        ]
        lane = lax.broadcasted_iota(jnp.int32, (L,), 0)

        def stage(w, b):
            idx_s, idx_d, rows_s, rows_d, sem_s, sem_d = bufs[b]
            base = pl.multiple_of(tile_base + w * W, W)
            pltpu.sync_copy(src_hbm.at[pl.ds(base, W)], idx_s)
            pltpu.sync_copy(dst_hbm.at[pl.ds(base, W)], idx_d)
            pltpu.async_copy(z_hbm.at[idx_s], rows_s, sem_s)
            pltpu.async_copy(z_hbm.at[idx_d], rows_d, sem_d)

        def wait(b):
            idx_s, idx_d, rows_s, rows_d, sem_s, sem_d = bufs[b]
            pltpu.make_async_copy(z_hbm.at[idx_s], rows_s, sem_s).wait()
            pltpu.make_async_copy(z_hbm.at[idx_d], rows_d, sem_d).wait()

        def compute(w, b):
            _, _, rows_s, rows_d, _, _ = bufs[b]
            base = pl.multiple_of(tile_base + w * W, W)

            @pl.loop(0, W // L)
            def _group(g):
                e0 = pl.multiple_of(g * L, L)

                def body(k, res):
                    e = e0 + k
                    acc = jnp.zeros((L,), jnp.float32)
                    for j in range(D_FEAT // L):
                        vs = rows_s[e, pl.ds(j * L, L)]
                        vd = rows_d[e, pl.ds(j * L, L)]
                        df = vs - vd
                        acc = acc + df * df
                    return jnp.where(lane == k, jnp.sum(acc), res)

                d2 = lax.fori_loop(0, L, body, jnp.zeros((L,), jnp.float32),
                                   unroll=4)
                out_v[pl.ds(e0, L)] = jnp.exp(-_sqrt_vec(d2))

            pltpu.sync_copy(out_v, out_hbm.at[pl.ds(base, W)])

        stage(0, 0)

        @pl.loop(0, (NWIN - 1) // 2)
        def _pair(p):
            w = p * 2
            stage(w + 1, 1)
            wait(0)
            compute(w, 0)
            stage(w + 2, 0)
            wait(1)
            compute(w + 1, 1)

        wait(0)
        compute(NWIN - 1, 0)

    return sc_kernel(z, src, dst)


# speedup vs baseline: 3.0707x; 1.3739x over previous
import dataclasses
import functools

import jax
import jax.numpy as jnp
from jax import lax
from jax.experimental import pallas as pl
from jax.experimental.pallas import tpu as pltpu
from jax.experimental.pallas import tpu_sc as plsc

N_NODES = 10000
N_EDGES = 320000
D_FEAT = 128
L = 16
NUM_CORES = 2
NUM_SUBCORES = 16
NW = NUM_CORES * NUM_SUBCORES
E_PER_TILE = N_EDGES // NW
W = 80
NWIN = E_PER_TILE // W


def _sqrt_vec(x):
    bits = plsc.bitcast(x, jnp.int32)
    y = plsc.bitcast(jnp.int32(0x5F3759DF) - (bits >> 1), jnp.float32)
    half = x * 0.5
    for _ in range(3):
        y = y * (1.5 - half * y * y)
    return x * y


def kernel(z, edge_index):
    src = edge_index[0]
    dst = edge_index[1]
    mesh = plsc.VectorSubcoreMesh(core_axis_name="c", subcore_axis_name="s")
    cp = pltpu.CompilerParams()
    if "needs_layout_passes" in pltpu.CompilerParams.__dataclass_fields__:
        cp = dataclasses.replace(cp, needs_layout_passes=False)

    @functools.partial(
        pl.kernel,
        out_type=jax.ShapeDtypeStruct((N_EDGES,), jnp.float32),
        mesh=mesh,
        compiler_params=cp,
        scratch_types=[
            pltpu.VMEM((W,), jnp.int32), pltpu.VMEM((W,), jnp.int32),
            pltpu.VMEM((W,), jnp.int32), pltpu.VMEM((W,), jnp.int32),
            pltpu.VMEM((W, D_FEAT), jnp.float32),
            pltpu.VMEM((W, D_FEAT), jnp.float32),
            pltpu.VMEM((W, D_FEAT), jnp.float32),
            pltpu.VMEM((W, D_FEAT), jnp.float32),
            pltpu.VMEM((W,), jnp.float32), pltpu.VMEM((W,), jnp.float32),
            pltpu.SemaphoreType.DMA, pltpu.SemaphoreType.DMA,
            pltpu.SemaphoreType.DMA, pltpu.SemaphoreType.DMA,
            pltpu.SemaphoreType.DMA, pltpu.SemaphoreType.DMA,
        ],
    )
    def sc_kernel(z_hbm, src_hbm, dst_hbm, out_hbm,
                  idx_s0, idx_d0, idx_s1, idx_d1,
                  rows_s0, rows_d0, rows_s1, rows_d1,
                  out_v0, out_v1,
                  sem_i0, sem_i1, sem_r0, sem_r1, sem_o0, sem_o1):
        wid = lax.axis_index("s") * NUM_CORES + lax.axis_index("c")
        tile_base = wid * E_PER_TILE
        idx_bufs = [(idx_s0, idx_d0, sem_i0), (idx_s1, idx_d1, sem_i1)]
        row_bufs = [(rows_s0, rows_d0, sem_r0), (rows_s1, rows_d1, sem_r1)]
        out_bufs = [(out_v0, sem_o0), (out_v1, sem_o1)]
        lane = lax.broadcasted_iota(jnp.int32, (L,), 0)

        def base_of(w):
            return pl.multiple_of(tile_base + w * W, W)

        def stage_idx(w, b):
            idx_s, idx_d, sem = idx_bufs[b]
            base = base_of(w)
            pltpu.async_copy(src_hbm.at[pl.ds(base, W)], idx_s, sem)
            pltpu.async_copy(dst_hbm.at[pl.ds(base, W)], idx_d, sem)

        def wait_idx(b):
            idx_s, idx_d, sem = idx_bufs[b]
            pltpu.make_async_copy(src_hbm.at[pl.ds(0, W)], idx_s, sem).wait()
            pltpu.make_async_copy(dst_hbm.at[pl.ds(0, W)], idx_d, sem).wait()

        def stage_rows(b):
            idx_s, idx_d, _ = idx_bufs[b]
            rows_s, rows_d, sem = row_bufs[b]
            wait_idx(b)
            pltpu.async_copy(z_hbm.at[idx_s], rows_s, sem)
            pltpu.async_copy(z_hbm.at[idx_d], rows_d, sem)

        def wait_rows(b):
            idx_s, idx_d, _ = idx_bufs[b]
            rows_s, rows_d, sem = row_bufs[b]
            pltpu.make_async_copy(z_hbm.at[idx_s], rows_s, sem).wait()
            pltpu.make_async_copy(z_hbm.at[idx_d], rows_d, sem).wait()

        def wait_out(b):
            out_v, sem = out_bufs[b]
            pltpu.make_async_copy(
                out_v, out_hbm.at[pl.ds(0, W)], sem).wait()

        def compute(w, b, drain_prev):
            rows_s, rows_d, _ = row_bufs[b]
            out_v, sem = out_bufs[b]
            if drain_prev:
                @pl.when(w >= 2)
                def _drain():
                    wait_out(b)

            @pl.loop(0, W // L)
            def _group(g):
                e0 = pl.multiple_of(g * L, L)

                @plsc.parallel_loop(0, L, 1, unroll=4,
                                    carry=jnp.zeros((L,), jnp.float32))
                def d2(k, res):
                    e = e0 + k
                    acc = jnp.zeros((L,), jnp.float32)
                    for j in range(D_FEAT // L):
                        vs = rows_s[e, pl.ds(j * L, L)]
                        vd = rows_d[e, pl.ds(j * L, L)]
                        df = vs - vd
                        acc = acc + df * df
                    return jnp.where(lane == k, jnp.sum(acc), res)

                out_v[pl.ds(e0, L)] = jnp.exp(-_sqrt_vec(d2))

            pltpu.async_copy(out_v, out_hbm.at[pl.ds(base_of(w), W)], sem)

        stage_idx(0, 0)
        stage_idx(1, 1)
        stage_rows(0)

        @pl.loop(0, (NWIN - 1) // 2)
        def _pair(p):
            w = p * 2
            stage_rows(1)
            wait_rows(0)
            stage_idx(w + 2, 0)
            compute(w, 0, drain_prev=True)
            wait_rows(1)
            stage_idx(jnp.minimum(w + 3, NWIN - 1), 1)
            stage_rows(0)
            compute(w + 1, 1, drain_prev=True)

        wait_rows(0)
        compute(NWIN - 1, 0, drain_prev=True)
        wait_out(0)
        wait_out(1)
        wait_idx(1)

    return sc_kernel(z, src, dst)


# speedup vs baseline: 3.3971x; 1.1063x over previous
import dataclasses
import functools

import jax
import jax.numpy as jnp
from jax import lax
from jax.experimental import pallas as pl
from jax.experimental.pallas import tpu as pltpu
from jax.experimental.pallas import tpu_sc as plsc

N_NODES = 10000
N_EDGES = 320000
D_FEAT = 128
L = 16
NUM_CORES = 2
NUM_SUBCORES = 16
NW = NUM_CORES * NUM_SUBCORES
E_PER_TILE = N_EDGES // NW
W = 80
NWIN = E_PER_TILE // W


def _sqrt_vec(x):
    bits = plsc.bitcast(x, jnp.int32)
    y = plsc.bitcast(jnp.int32(0x5F3759DF) - (bits >> 1), jnp.float32)
    half = x * 0.5
    for _ in range(3):
        y = y * (1.5 - half * y * y)
    return x * y


def kernel(z, edge_index):
    src = edge_index[0]
    dst = edge_index[1]
    mesh = plsc.VectorSubcoreMesh(core_axis_name="c", subcore_axis_name="s")
    cp = pltpu.CompilerParams()
    if "needs_layout_passes" in pltpu.CompilerParams.__dataclass_fields__:
        cp = dataclasses.replace(cp, needs_layout_passes=False)
    if "use_tc_tiling_on_sc" in pltpu.CompilerParams.__dataclass_fields__:
        cp = dataclasses.replace(cp, use_tc_tiling_on_sc=False)

    @functools.partial(
        pl.kernel,
        out_type=jax.ShapeDtypeStruct((N_EDGES,), jnp.float32),
        mesh=mesh,
        compiler_params=cp,
        scratch_types=[
            pltpu.VMEM((W,), jnp.int32), pltpu.VMEM((W,), jnp.int32),
            pltpu.VMEM((W,), jnp.int32), pltpu.VMEM((W,), jnp.int32),
            pltpu.VMEM((W, D_FEAT // 2), jnp.int32),
            pltpu.VMEM((W, D_FEAT // 2), jnp.int32),
            pltpu.VMEM((W, D_FEAT // 2), jnp.int32),
            pltpu.VMEM((W, D_FEAT // 2), jnp.int32),
            pltpu.VMEM((W,), jnp.float32), pltpu.VMEM((W,), jnp.float32),
            pltpu.SemaphoreType.DMA, pltpu.SemaphoreType.DMA,
            pltpu.SemaphoreType.DMA, pltpu.SemaphoreType.DMA,
            pltpu.SemaphoreType.DMA, pltpu.SemaphoreType.DMA,
        ],
    )
    def sc_kernel(z_hbm, src_hbm, dst_hbm, out_hbm,
                  idx_s0, idx_d0, idx_s1, idx_d1,
                  rows_s0, rows_d0, rows_s1, rows_d1,
                  out_v0, out_v1,
                  sem_i0, sem_i1, sem_r0, sem_r1, sem_o0, sem_o1):
        wid = lax.axis_index("s") * NUM_CORES + lax.axis_index("c")
        tile_base = wid * E_PER_TILE
        idx_bufs = [(idx_s0, idx_d0, sem_i0), (idx_s1, idx_d1, sem_i1)]
        row_bufs = [(rows_s0, rows_d0, sem_r0), (rows_s1, rows_d1, sem_r1)]
        out_bufs = [(out_v0, sem_o0), (out_v1, sem_o1)]
        lane = lax.broadcasted_iota(jnp.int32, (L,), 0)

        def base_of(w):
            return pl.multiple_of(tile_base + w * W, W)

        def stage_idx(w, b):
            idx_s, idx_d, sem = idx_bufs[b]
            base = base_of(w)
            pltpu.async_copy(src_hbm.at[pl.ds(base, W)], idx_s, sem)
            pltpu.async_copy(dst_hbm.at[pl.ds(base, W)], idx_d, sem)

        def wait_idx(b):
            idx_s, idx_d, sem = idx_bufs[b]
            pltpu.make_async_copy(src_hbm.at[pl.ds(0, W)], idx_s, sem).wait()
            pltpu.make_async_copy(dst_hbm.at[pl.ds(0, W)], idx_d, sem).wait()

        def stage_rows(b):
            idx_s, idx_d, _ = idx_bufs[b]
            rows_s, rows_d, sem = row_bufs[b]
            wait_idx(b)
            pltpu.async_copy(z_hbm.at[idx_s], rows_s, sem)
            pltpu.async_copy(z_hbm.at[idx_d], rows_d, sem)

        def wait_rows(b):
            idx_s, idx_d, _ = idx_bufs[b]
            rows_s, rows_d, sem = row_bufs[b]
            pltpu.make_async_copy(z_hbm.at[idx_s], rows_s, sem).wait()
            pltpu.make_async_copy(z_hbm.at[idx_d], rows_d, sem).wait()

        def wait_out(b):
            out_v, sem = out_bufs[b]
            pltpu.make_async_copy(
                out_v, out_hbm.at[pl.ds(0, W)], sem).wait()

        def compute(w, b, drain_prev):
            rows_s, rows_d, _ = row_bufs[b]
            out_v, sem = out_bufs[b]
            if drain_prev:
                @pl.when(w >= 2)
                def _drain():
                    wait_out(b)

            @pl.loop(0, W // L)
            def _group(g):
                e0 = pl.multiple_of(g * L, L)

                @plsc.parallel_loop(0, L, 1, unroll=4,
                                    carry=jnp.zeros((L,), jnp.float32))
                def d2(k, res):
                    e = e0 + k
                    acc = jnp.zeros((L,), jnp.float32)
                    for j in range(D_FEAT // (2 * L)):
                        vs = plsc.bitcast(rows_s[e, pl.ds(j * L, L)],
                                          jnp.bfloat16)
                        vd = plsc.bitcast(rows_d[e, pl.ds(j * L, L)],
                                          jnp.bfloat16)
                        df = vs - vd
                        lo, hi = plsc.unpack(
                            df, format=plsc.PackFormat.INTERLEAVED)
                        acc = acc + lo * lo
                        acc = acc + hi * hi
                    return jnp.where(lane == k, jnp.sum(acc), res)

                out_v[pl.ds(e0, L)] = jnp.exp(-_sqrt_vec(d2))

            pltpu.async_copy(out_v, out_hbm.at[pl.ds(base_of(w), W)], sem)

        stage_idx(0, 0)
        stage_idx(1, 1)
        stage_rows(0)

        @pl.loop(0, (NWIN - 1) // 2)
        def _pair(p):
            w = p * 2
            stage_rows(1)
            wait_rows(0)
            stage_idx(w + 2, 0)
            compute(w, 0, drain_prev=True)
            wait_rows(1)
            stage_idx(jnp.minimum(w + 3, NWIN - 1), 1)
            stage_rows(0)
            compute(w + 1, 1, drain_prev=True)

        wait_rows(0)
        compute(NWIN - 1, 0, drain_prev=True)
        wait_out(0)
        wait_out(1)
        wait_idx(1)

    z_packed = lax.bitcast_convert_type(
        z.astype(jnp.bfloat16).reshape(N_NODES, D_FEAT // 2, 2), jnp.int32)
    return sc_kernel(z_packed, src, dst)


# speedup vs baseline: 3.8737x; 1.1403x over previous
import dataclasses
import functools

import jax
import jax.numpy as jnp
from jax import lax
from jax.experimental import pallas as pl
from jax.experimental.pallas import tpu as pltpu
from jax.experimental.pallas import tpu_sc as plsc

N_NODES = 10000
N_EDGES = 320000
D_FEAT = 128
D_PK = D_FEAT // 2
L = 16
NUM_CORES = 2
NUM_SUBCORES = 16
NW = NUM_CORES * NUM_SUBCORES
E_PER_TILE = N_EDGES // NW
W = 80
NWIN = E_PER_TILE // W
R_PER_TILE = N_NODES // NUM_SUBCORES
R_CHUNK = 125


def _sqrt_vec(x):
    bits = plsc.bitcast(x, jnp.int32)
    y = plsc.bitcast(jnp.int32(0x5F3759DF) - (bits >> 1), jnp.float32)
    half = x * 0.5
    for _ in range(3):
        y = y * (1.5 - half * y * y)
    return x * y


def kernel(z, edge_index):
    mesh = plsc.VectorSubcoreMesh(core_axis_name="c", subcore_axis_name="s")
    cp = pltpu.CompilerParams()
    if "needs_layout_passes" in pltpu.CompilerParams.__dataclass_fields__:
        cp = dataclasses.replace(cp, needs_layout_passes=False)
    if "use_tc_tiling_on_sc" in pltpu.CompilerParams.__dataclass_fields__:
        cp = dataclasses.replace(cp, use_tc_tiling_on_sc=False)

    @functools.partial(
        pl.kernel,
        out_type=(
            jax.ShapeDtypeStruct((N_EDGES,), jnp.float32),
            jax.ShapeDtypeStruct((N_NODES, D_PK), jnp.int32),
        ),
        mesh=mesh,
        compiler_params=cp,
        scratch_types=[
            pltpu.VMEM((R_CHUNK, D_FEAT), jnp.float32),
            pltpu.VMEM((R_CHUNK, D_PK), jnp.int32),
            pltpu.VMEM((W,), jnp.int32), pltpu.VMEM((W,), jnp.int32),
            pltpu.VMEM((W,), jnp.int32), pltpu.VMEM((W,), jnp.int32),
            pltpu.VMEM((W, D_PK), jnp.int32),
            pltpu.VMEM((W, D_PK), jnp.int32),
            pltpu.VMEM((W, D_PK), jnp.int32),
            pltpu.VMEM((W, D_PK), jnp.int32),
            pltpu.VMEM((W,), jnp.float32), pltpu.VMEM((W,), jnp.float32),
            pltpu.SemaphoreType.DMA, pltpu.SemaphoreType.DMA,
            pltpu.SemaphoreType.DMA, pltpu.SemaphoreType.DMA,
            pltpu.SemaphoreType.DMA, pltpu.SemaphoreType.DMA,
        ],
    )
    def sc_kernel(z_hbm, ei_hbm, out_hbm, zpk_hbm,
                  pin, pout,
                  idx_s0, idx_d0, idx_s1, idx_d1,
                  rows_s0, rows_d0, rows_s1, rows_d1,
                  out_v0, out_v1,
                  sem_i0, sem_i1, sem_r0, sem_r1, sem_o0, sem_o1):
        cid = lax.axis_index("c")
        sid = lax.axis_index("s")
        wid = sid * NUM_CORES + cid
        tile_base = wid * E_PER_TILE
        idx_bufs = [(idx_s0, idx_d0, sem_i0), (idx_s1, idx_d1, sem_i1)]
        row_bufs = [(rows_s0, rows_d0, sem_r0), (rows_s1, rows_d1, sem_r1)]
        out_bufs = [(out_v0, sem_o0), (out_v1, sem_o1)]
        lane = lax.broadcasted_iota(jnp.int32, (L,), 0)

        def base_of(w):
            return pl.multiple_of(tile_base + w * W, W)

        def stage_idx(w, b):
            idx_s, idx_d, sem = idx_bufs[b]
            base = base_of(w)
            pltpu.async_copy(ei_hbm.at[0, pl.ds(base, W)], idx_s, sem)
            pltpu.async_copy(ei_hbm.at[1, pl.ds(base, W)], idx_d, sem)

        def wait_idx(b):
            idx_s, idx_d, sem = idx_bufs[b]
            pltpu.make_async_copy(ei_hbm.at[0, pl.ds(0, W)], idx_s, sem).wait()
            pltpu.make_async_copy(ei_hbm.at[1, pl.ds(0, W)], idx_d, sem).wait()

        def stage_rows(b):
            idx_s, idx_d, _ = idx_bufs[b]
            rows_s, rows_d, sem = row_bufs[b]
            wait_idx(b)
            pltpu.async_copy(zpk_hbm.at[idx_s], rows_s, sem)
            pltpu.async_copy(zpk_hbm.at[idx_d], rows_d, sem)

        def wait_rows(b):
            idx_s, idx_d, _ = idx_bufs[b]
            rows_s, rows_d, sem = row_bufs[b]
            pltpu.make_async_copy(zpk_hbm.at[idx_s], rows_s, sem).wait()
            pltpu.make_async_copy(zpk_hbm.at[idx_d], rows_d, sem).wait()

        def wait_out(b):
            out_v, sem = out_bufs[b]
            pltpu.make_async_copy(
                out_v, out_hbm.at[pl.ds(0, W)], sem).wait()

        def compute(w, b, drain_prev):
            rows_s, rows_d, _ = row_bufs[b]
            out_v, sem = out_bufs[b]
            if drain_prev:
                @pl.when(w >= 2)
                def _drain():
                    wait_out(b)

            @pl.loop(0, W // L)
            def _group(g):
                e0 = pl.multiple_of(g * L, L)

                @plsc.parallel_loop(0, L, 1, unroll=4,
                                    carry=jnp.zeros((L,), jnp.float32))
                def d2(k, res):
                    e = e0 + k
                    acc = jnp.zeros((L,), jnp.float32)
                    for j in range(D_PK // L):
                        vs = plsc.bitcast(rows_s[e, pl.ds(j * L, L)],
                                          jnp.bfloat16)
                        vd = plsc.bitcast(rows_d[e, pl.ds(j * L, L)],
                                          jnp.bfloat16)
                        df = vs - vd
                        lo, hi = plsc.unpack(
                            df, format=plsc.PackFormat.INTERLEAVED)
                        acc = acc + lo * lo
                        acc = acc + hi * hi
                    return jnp.where(lane == k, jnp.sum(acc), res)

                out_v[pl.ds(e0, L)] = jnp.exp(-_sqrt_vec(d2))

            pltpu.async_copy(out_v, out_hbm.at[pl.ds(base_of(w), W)], sem)

        stage_idx(0, 0)
        stage_idx(1, 1)

        r0 = sid * R_PER_TILE

        @pl.loop(0, R_PER_TILE // R_CHUNK)
        def _chunk(ci):
            rbase = r0 + ci * R_CHUNK
            pltpu.sync_copy(z_hbm.at[pl.ds(rbase, R_CHUNK)], pin)

            @pl.loop(0, R_CHUNK)
            def _row(r):
                for j in range(D_PK // L):
                    a = pin[r, pl.ds(j * 2 * L, L)]
                    b = pin[r, pl.ds(j * 2 * L + L, L)]
                    pk = plsc.pack(a, b, format=plsc.PackFormat.INTERLEAVED)
                    pout[r, pl.ds(j * L, L)] = plsc.bitcast(pk, jnp.int32)

            pltpu.sync_copy(pout, zpk_hbm.at[pl.ds(rbase, R_CHUNK)])

        plsc.subcore_barrier()

        stage_rows(0)

        @pl.loop(0, (NWIN - 1) // 2)
        def _pair(p):
            w = p * 2
            stage_rows(1)
            wait_rows(0)
            stage_idx(w + 2, 0)
            compute(w, 0, drain_prev=True)
            wait_rows(1)
            stage_idx(jnp.minimum(w + 3, NWIN - 1), 1)
            stage_rows(0)
            compute(w + 1, 1, drain_prev=True)

        wait_rows(0)
        compute(NWIN - 1, 0, drain_prev=True)
        wait_out(0)
        wait_out(1)
        wait_idx(1)

    probs, _ = sc_kernel(z, edge_index)
    return probs


# speedup vs baseline: 4.0449x; 1.0442x over previous
import dataclasses
import functools

import jax
import jax.numpy as jnp
from jax import lax
from jax.experimental import pallas as pl
from jax.experimental.pallas import tpu as pltpu
from jax.experimental.pallas import tpu_sc as plsc

N_NODES = 10000
N_EDGES = 320000
D_FEAT = 128
D_PK = D_FEAT // 2
L = 16
NUM_CORES = 2
NUM_SUBCORES = 16
NW = NUM_CORES * NUM_SUBCORES
E_PER_TILE = N_EDGES // NW
W = 80
NWIN = E_PER_TILE // W
R_PER_TILE = N_NODES // NUM_SUBCORES
R_CHUNK = 125


def _sqrt_vec(x):
    bits = plsc.bitcast(x, jnp.int32)
    y = plsc.bitcast(jnp.int32(0x5F3759DF) - (bits >> 1), jnp.float32)
    half = x * 0.5
    for _ in range(3):
        y = y * (1.5 - half * y * y)
    return x * y


def kernel(z, edge_index):
    mesh = plsc.VectorSubcoreMesh(core_axis_name="c", subcore_axis_name="s")
    cp = pltpu.CompilerParams()
    if "needs_layout_passes" in pltpu.CompilerParams.__dataclass_fields__:
        cp = dataclasses.replace(cp, needs_layout_passes=False)
    if "use_tc_tiling_on_sc" in pltpu.CompilerParams.__dataclass_fields__:
        cp = dataclasses.replace(cp, use_tc_tiling_on_sc=False)

    @functools.partial(
        pl.kernel,
        out_type=(
            jax.ShapeDtypeStruct((N_EDGES,), jnp.float32),
            jax.ShapeDtypeStruct((N_NODES, D_PK), jnp.int32),
        ),
        mesh=mesh,
        compiler_params=cp,
        scratch_types=[
            pltpu.VMEM((R_CHUNK, D_FEAT), jnp.float32),
            pltpu.VMEM((R_CHUNK, D_FEAT), jnp.float32),
            pltpu.VMEM((R_CHUNK, D_PK), jnp.int32),
            pltpu.VMEM((R_CHUNK, D_PK), jnp.int32),
            pltpu.SemaphoreType.DMA, pltpu.SemaphoreType.DMA,
            pltpu.SemaphoreType.DMA, pltpu.SemaphoreType.DMA,
            pltpu.VMEM((W,), jnp.int32), pltpu.VMEM((W,), jnp.int32),
            pltpu.VMEM((W,), jnp.int32), pltpu.VMEM((W,), jnp.int32),
            pltpu.VMEM((W, D_PK), jnp.int32),
            pltpu.VMEM((W, D_PK), jnp.int32),
            pltpu.VMEM((W, D_PK), jnp.int32),
            pltpu.VMEM((W, D_PK), jnp.int32),
            pltpu.VMEM((W,), jnp.float32), pltpu.VMEM((W,), jnp.float32),
            pltpu.SemaphoreType.DMA, pltpu.SemaphoreType.DMA,
            pltpu.SemaphoreType.DMA, pltpu.SemaphoreType.DMA,
            pltpu.SemaphoreType.DMA, pltpu.SemaphoreType.DMA,
        ],
    )
    def sc_kernel(z_hbm, ei_hbm, out_hbm, zpk_hbm,
                  pin0, pin1, pout0, pout1,
                  sem_pi0, sem_pi1, sem_po0, sem_po1,
                  idx_s0, idx_d0, idx_s1, idx_d1,
                  rows_s0, rows_d0, rows_s1, rows_d1,
                  out_v0, out_v1,
                  sem_i0, sem_i1, sem_r0, sem_r1, sem_o0, sem_o1):
        cid = lax.axis_index("c")
        sid = lax.axis_index("s")
        wid = sid * NUM_CORES + cid
        tile_base = wid * E_PER_TILE
        idx_bufs = [(idx_s0, idx_d0, sem_i0), (idx_s1, idx_d1, sem_i1)]
        row_bufs = [(rows_s0, rows_d0, sem_r0), (rows_s1, rows_d1, sem_r1)]
        out_bufs = [(out_v0, sem_o0), (out_v1, sem_o1)]
        lane = lax.broadcasted_iota(jnp.int32, (L,), 0)

        def base_of(w):
            return pl.multiple_of(tile_base + w * W, W)

        def stage_idx(w, b):
            idx_s, idx_d, sem = idx_bufs[b]
            base = base_of(w)
            pltpu.async_copy(ei_hbm.at[0, pl.ds(base, W)], idx_s, sem)
            pltpu.async_copy(ei_hbm.at[1, pl.ds(base, W)], idx_d, sem)

        def wait_idx(b):
            idx_s, idx_d, sem = idx_bufs[b]
            pltpu.make_async_copy(ei_hbm.at[0, pl.ds(0, W)], idx_s, sem).wait()
            pltpu.make_async_copy(ei_hbm.at[1, pl.ds(0, W)], idx_d, sem).wait()

        def stage_rows(b):
            idx_s, idx_d, _ = idx_bufs[b]
            rows_s, rows_d, sem = row_bufs[b]
            wait_idx(b)
            pltpu.async_copy(zpk_hbm.at[idx_s], rows_s, sem)
            pltpu.async_copy(zpk_hbm.at[idx_d], rows_d, sem)

        def wait_rows(b):
            idx_s, idx_d, _ = idx_bufs[b]
            rows_s, rows_d, sem = row_bufs[b]
            pltpu.make_async_copy(zpk_hbm.at[idx_s], rows_s, sem).wait()
            pltpu.make_async_copy(zpk_hbm.at[idx_d], rows_d, sem).wait()

        def wait_out(b):
            out_v, sem = out_bufs[b]
            pltpu.make_async_copy(
                out_v, out_hbm.at[pl.ds(0, W)], sem).wait()

        def compute(w, b, drain_prev):
            rows_s, rows_d, _ = row_bufs[b]
            out_v, sem = out_bufs[b]
            if drain_prev:
                @pl.when(w >= 2)
                def _drain():
                    wait_out(b)

            @pl.loop(0, W // L)
            def _group(g):
                e0 = pl.multiple_of(g * L, L)

                @plsc.parallel_loop(0, L, 1, unroll=4,
                                    carry=jnp.zeros((L,), jnp.float32))
                def d2(k, res):
                    e = e0 + k
                    acc = jnp.zeros((L,), jnp.float32)
                    for j in range(D_PK // L):
                        vs = plsc.bitcast(rows_s[e, pl.ds(j * L, L)],
                                          jnp.bfloat16)
                        vd = plsc.bitcast(rows_d[e, pl.ds(j * L, L)],
                                          jnp.bfloat16)
                        df = vs - vd
                        lo, hi = plsc.unpack(
                            df, format=plsc.PackFormat.INTERLEAVED)
                        acc = acc + lo * lo
                        acc = acc + hi * hi
                    return jnp.where(lane == k, jnp.sum(acc), res)

                out_v[pl.ds(e0, L)] = jnp.exp(-_sqrt_vec(d2))

            pltpu.async_copy(out_v, out_hbm.at[pl.ds(base_of(w), W)], sem)

        stage_idx(0, 0)
        stage_idx(1, 1)

        r0 = sid * R_PER_TILE
        pins = [(pin0, sem_pi0), (pin1, sem_pi1)]
        pouts = [(pout0, sem_po0), (pout1, sem_po1)]
        n_chunks = R_PER_TILE // R_CHUNK

        def chunk_in(ci, b):
            pin, sem = pins[b]
            pltpu.async_copy(
                z_hbm.at[pl.ds(r0 + ci * R_CHUNK, R_CHUNK)], pin, sem)

        def chunk_in_wait(b):
            pin, sem = pins[b]
            pltpu.make_async_copy(
                z_hbm.at[pl.ds(0, R_CHUNK)], pin, sem).wait()

        def chunk_out_wait(b):
            pout, sem = pouts[b]
            pltpu.make_async_copy(
                pout, zpk_hbm.at[pl.ds(0, R_CHUNK)], sem).wait()

        chunk_in(0, 0)
        for ci in range(n_chunks):
            b = ci % 2
            if ci + 1 < n_chunks:
                chunk_in(ci + 1, 1 - b)
            chunk_in_wait(b)
            if ci >= 2:
                chunk_out_wait(b)
            pin, _ = pins[b]
            pout, sem_po = pouts[b]

            @pl.loop(0, R_CHUNK)
            def _row(r):
                for j in range(D_PK // L):
                    a = pin[r, pl.ds(j * 2 * L, L)]
                    bb = pin[r, pl.ds(j * 2 * L + L, L)]
                    pk = plsc.pack(a, bb, format=plsc.PackFormat.INTERLEAVED)
                    pout[r, pl.ds(j * L, L)] = plsc.bitcast(pk, jnp.int32)

            pltpu.async_copy(
                pout, zpk_hbm.at[pl.ds(r0 + ci * R_CHUNK, R_CHUNK)], sem_po)

        chunk_out_wait((n_chunks - 2) % 2)
        chunk_out_wait((n_chunks - 1) % 2)
        plsc.subcore_barrier()

        stage_rows(0)

        @pl.loop(0, (NWIN - 1) // 2)
        def _pair(p):
            w = p * 2
            stage_rows(1)
            wait_rows(0)
            stage_idx(w + 2, 0)
            compute(w, 0, drain_prev=True)
            wait_rows(1)
            stage_idx(jnp.minimum(w + 3, NWIN - 1), 1)
            stage_rows(0)
            compute(w + 1, 1, drain_prev=True)

        wait_rows(0)
        compute(NWIN - 1, 0, drain_prev=True)
        wait_out(0)
        wait_out(1)
        wait_idx(1)

    probs, _ = sc_kernel(z, edge_index)
    return probs


# speedup vs baseline: 4.1874x; 1.0352x over previous
import dataclasses
import functools

import jax
import jax.numpy as jnp
from jax import lax
from jax.experimental import pallas as pl
from jax.experimental.pallas import tpu as pltpu
from jax.experimental.pallas import tpu_sc as plsc

N_NODES = 10000
N_EDGES = 320000
D_FEAT = 128
D_PK = D_FEAT // 2
L = 16
NUM_CORES = 2
NUM_SUBCORES = 16
NW = NUM_CORES * NUM_SUBCORES
E_PER_TILE = N_EDGES // NW
W = 80
NWIN = E_PER_TILE // W
R_PER_TILE = N_NODES // NUM_SUBCORES
R_CHUNK = 125


def _sqrt_vec(x):
    bits = plsc.bitcast(x, jnp.int32)
    y = plsc.bitcast(jnp.int32(0x5F3759DF) - (bits >> 1), jnp.float32)
    half = x * 0.5
    for _ in range(3):
        y = y * (1.5 - half * y * y)
    return x * y


def kernel(z, edge_index):
    mesh = plsc.VectorSubcoreMesh(core_axis_name="c", subcore_axis_name="s")
    cp = pltpu.CompilerParams()
    if "needs_layout_passes" in pltpu.CompilerParams.__dataclass_fields__:
        cp = dataclasses.replace(cp, needs_layout_passes=False)
    if "use_tc_tiling_on_sc" in pltpu.CompilerParams.__dataclass_fields__:
        cp = dataclasses.replace(cp, use_tc_tiling_on_sc=False)

    @functools.partial(
        pl.kernel,
        out_type=(
            jax.ShapeDtypeStruct((N_EDGES,), jnp.float32),
            jax.ShapeDtypeStruct((N_NODES, D_PK), jnp.int32),
        ),
        mesh=mesh,
        compiler_params=cp,
        scratch_types=[
            pltpu.VMEM((R_CHUNK, D_FEAT), jnp.float32),
            pltpu.VMEM((R_CHUNK, D_FEAT), jnp.float32),
            pltpu.VMEM((R_CHUNK, D_PK), jnp.int32),
            pltpu.VMEM((R_CHUNK, D_PK), jnp.int32),
            pltpu.SemaphoreType.DMA, pltpu.SemaphoreType.DMA,
            pltpu.SemaphoreType.DMA, pltpu.SemaphoreType.DMA,
            pltpu.VMEM((W,), jnp.int32), pltpu.VMEM((W,), jnp.int32),
            pltpu.VMEM((W,), jnp.int32), pltpu.VMEM((W,), jnp.int32),
            pltpu.VMEM((W, D_PK), jnp.int32),
            pltpu.VMEM((W, D_PK), jnp.int32),
            pltpu.VMEM((W, D_PK), jnp.int32),
            pltpu.VMEM((W, D_PK), jnp.int32),
            pltpu.VMEM((W,), jnp.float32), pltpu.VMEM((W,), jnp.float32),
            pltpu.SemaphoreType.DMA, pltpu.SemaphoreType.DMA,
            pltpu.SemaphoreType.DMA, pltpu.SemaphoreType.DMA,
            pltpu.SemaphoreType.DMA, pltpu.SemaphoreType.DMA,
        ],
    )
    def sc_kernel(z_hbm, ei_hbm, out_hbm, zpk_hbm,
                  pin0, pin1, pout0, pout1,
                  sem_pi0, sem_pi1, sem_po0, sem_po1,
                  idx_s0, idx_d0, idx_s1, idx_d1,
                  rows_s0, rows_d0, rows_s1, rows_d1,
                  out_v0, out_v1,
                  sem_i0, sem_i1, sem_r0, sem_r1, sem_o0, sem_o1):
        cid = lax.axis_index("c")
        sid = lax.axis_index("s")
        wid = sid * NUM_CORES + cid
        tile_base = wid * E_PER_TILE
        idx_bufs = [(idx_s0, idx_d0, sem_i0), (idx_s1, idx_d1, sem_i1)]
        row_bufs = [(rows_s0, rows_d0, sem_r0), (rows_s1, rows_d1, sem_r1)]
        out_bufs = [(out_v0, sem_o0), (out_v1, sem_o1)]
        lane = lax.broadcasted_iota(jnp.int32, (L,), 0)

        def base_of(w):
            return pl.multiple_of(tile_base + w * W, W)

        def stage_idx(w, b):
            idx_s, idx_d, sem = idx_bufs[b]
            base = base_of(w)
            pltpu.async_copy(ei_hbm.at[0, pl.ds(base, W)], idx_s, sem)
            pltpu.async_copy(ei_hbm.at[1, pl.ds(base, W)], idx_d, sem)

        def wait_idx(b):
            idx_s, idx_d, sem = idx_bufs[b]
            pltpu.make_async_copy(ei_hbm.at[0, pl.ds(0, W)], idx_s, sem).wait()
            pltpu.make_async_copy(ei_hbm.at[1, pl.ds(0, W)], idx_d, sem).wait()

        def stage_rows(b):
            idx_s, idx_d, _ = idx_bufs[b]
            rows_s, rows_d, sem = row_bufs[b]
            wait_idx(b)
            pltpu.async_copy(zpk_hbm.at[idx_s], rows_s, sem)
            pltpu.async_copy(zpk_hbm.at[idx_d], rows_d, sem)

        def wait_rows(b):
            idx_s, idx_d, _ = idx_bufs[b]
            rows_s, rows_d, sem = row_bufs[b]
            pltpu.make_async_copy(zpk_hbm.at[idx_s], rows_s, sem).wait()
            pltpu.make_async_copy(zpk_hbm.at[idx_d], rows_d, sem).wait()

        def wait_out(b):
            out_v, sem = out_bufs[b]
            pltpu.make_async_copy(
                out_v, out_hbm.at[pl.ds(0, W)], sem).wait()

        def compute(w, b, drain_prev):
            rows_s, rows_d, _ = row_bufs[b]
            out_v, sem = out_bufs[b]
            if drain_prev:
                @pl.when(w >= 2)
                def _drain():
                    wait_out(b)

            @pl.loop(0, W // L)
            def _group(g):
                e0 = pl.multiple_of(g * L, L)

                @plsc.parallel_loop(0, L, 1, unroll=4,
                                    carry=jnp.zeros((L,), jnp.float32))
                def d2(k, res):
                    e = e0 + k
                    acc = jnp.zeros((L,), jnp.float32)
                    for j in range(D_PK // L):
                        vs = plsc.bitcast(rows_s[e, pl.ds(j * L, L)],
                                          jnp.bfloat16)
                        vd = plsc.bitcast(rows_d[e, pl.ds(j * L, L)],
                                          jnp.bfloat16)
                        df = vs - vd
                        lo, hi = plsc.unpack(
                            df, format=plsc.PackFormat.INTERLEAVED)
                        acc = acc + lo * lo
                        acc = acc + hi * hi
                    return jnp.where(lane == k, jnp.sum(acc), res)

                out_v[pl.ds(e0, L)] = jnp.exp(-_sqrt_vec(d2))

            pltpu.async_copy(out_v, out_hbm.at[pl.ds(base_of(w), W)], sem)

        stage_idx(0, 0)
        stage_idx(1, 1)

        r0 = sid * R_PER_TILE
        pins = [(pin0, sem_pi0), (pin1, sem_pi1)]
        pouts = [(pout0, sem_po0), (pout1, sem_po1)]
        n_chunks = R_PER_TILE // R_CHUNK

        def chunk_in(ci, b):
            pin, sem = pins[b]
            pltpu.async_copy(
                z_hbm.at[pl.ds(r0 + ci * R_CHUNK, R_CHUNK)], pin, sem)

        def chunk_in_wait(b):
            pin, sem = pins[b]
            pltpu.make_async_copy(
                z_hbm.at[pl.ds(0, R_CHUNK)], pin, sem).wait()

        def chunk_out_wait(b):
            pout, sem = pouts[b]
            pltpu.make_async_copy(
                pout, zpk_hbm.at[pl.ds(0, R_CHUNK)], sem).wait()

        chunk_in(0, 0)
        for ci in range(n_chunks):
            b = ci % 2
            if ci + 1 < n_chunks:
                chunk_in(ci + 1, 1 - b)
            chunk_in_wait(b)
            if ci >= 2:
                chunk_out_wait(b)
            pin, _ = pins[b]
            pout, sem_po = pouts[b]

            @plsc.parallel_loop(0, R_CHUNK, 1, unroll=4)
            def _row(r):
                for j in range(D_PK // L):
                    a = pin[r, pl.ds(j * 2 * L, L)]
                    bb = pin[r, pl.ds(j * 2 * L + L, L)]
                    pk = plsc.pack(a, bb, format=plsc.PackFormat.INTERLEAVED)
                    pout[r, pl.ds(j * L, L)] = plsc.bitcast(pk, jnp.int32)

            pltpu.async_copy(
                pout, zpk_hbm.at[pl.ds(r0 + ci * R_CHUNK, R_CHUNK)], sem_po)

        chunk_out_wait((n_chunks - 2) % 2)
        chunk_out_wait((n_chunks - 1) % 2)
        plsc.subcore_barrier()

        stage_rows(0)

        @pl.loop(0, (NWIN - 1) // 2)
        def _pair(p):
            w = p * 2
            stage_rows(1)
            wait_rows(0)
            stage_idx(w + 2, 0)
            compute(w, 0, drain_prev=True)
            wait_rows(1)
            stage_idx(jnp.minimum(w + 3, NWIN - 1), 1)
            stage_rows(0)
            compute(w + 1, 1, drain_prev=True)

        wait_rows(0)
        compute(NWIN - 1, 0, drain_prev=True)
        wait_out(0)
        wait_out(1)
        wait_idx(1)

    probs, _ = sc_kernel(z, edge_index)
    return probs


# speedup vs baseline: 4.6058x; 1.0999x over previous
import dataclasses
import functools

import jax
import jax.numpy as jnp
from jax import lax
from jax.experimental import pallas as pl
from jax.experimental.pallas import tpu as pltpu
from jax.experimental.pallas import tpu_sc as plsc

N_NODES = 10000
N_EDGES = 320000
D_FEAT = 128
D_PK = D_FEAT // 2
L = 16
NUM_CORES = 2
NUM_SUBCORES = 16
NW = NUM_CORES * NUM_SUBCORES
E_PER_TILE = N_EDGES // NW
W = 128
NWIN = E_PER_TILE // W
TAIL = E_PER_TILE - NWIN * W
R_PER_TILE = N_NODES // NUM_SUBCORES
R_CHUNK = 125


def _sqrt_vec(x):
    bits = plsc.bitcast(x, jnp.int32)
    y = plsc.bitcast(jnp.int32(0x5F3759DF) - (bits >> 1), jnp.float32)
    half = x * 0.5
    for _ in range(3):
        y = y * (1.5 - half * y * y)
    return x * y


def kernel(z, edge_index):
    mesh = plsc.VectorSubcoreMesh(core_axis_name="c", subcore_axis_name="s")
    cp = pltpu.CompilerParams()
    if "needs_layout_passes" in pltpu.CompilerParams.__dataclass_fields__:
        cp = dataclasses.replace(cp, needs_layout_passes=False)
    if "use_tc_tiling_on_sc" in pltpu.CompilerParams.__dataclass_fields__:
        cp = dataclasses.replace(cp, use_tc_tiling_on_sc=False)

    @functools.partial(
        pl.kernel,
        out_type=(
            jax.ShapeDtypeStruct((N_EDGES,), jnp.float32),
            jax.ShapeDtypeStruct((N_NODES, D_PK), jnp.int32),
        ),
        mesh=mesh,
        compiler_params=cp,
        scratch_types=[
            pltpu.VMEM((R_CHUNK, D_FEAT), jnp.float32),
            pltpu.VMEM((R_CHUNK, D_FEAT), jnp.float32),
            pltpu.VMEM((R_CHUNK, D_PK), jnp.int32),
            pltpu.VMEM((R_CHUNK, D_PK), jnp.int32),
            pltpu.SemaphoreType.DMA, pltpu.SemaphoreType.DMA,
            pltpu.SemaphoreType.DMA, pltpu.SemaphoreType.DMA,
            pltpu.VMEM((W,), jnp.int32), pltpu.VMEM((W,), jnp.int32),
            pltpu.VMEM((W,), jnp.int32), pltpu.VMEM((W,), jnp.int32),
            pltpu.VMEM((W, D_PK), jnp.int32),
            pltpu.VMEM((W, D_PK), jnp.int32),
            pltpu.VMEM((W, D_PK), jnp.int32),
            pltpu.VMEM((W, D_PK), jnp.int32),
            pltpu.VMEM((W,), jnp.float32), pltpu.VMEM((W,), jnp.float32),
            pltpu.VMEM((TAIL,), jnp.int32), pltpu.VMEM((TAIL,), jnp.int32),
            pltpu.VMEM((TAIL, D_PK), jnp.int32),
            pltpu.VMEM((TAIL, D_PK), jnp.int32),
            pltpu.VMEM((TAIL,), jnp.float32),
            pltpu.SemaphoreType.DMA, pltpu.SemaphoreType.DMA,
            pltpu.SemaphoreType.DMA, pltpu.SemaphoreType.DMA,
            pltpu.SemaphoreType.DMA, pltpu.SemaphoreType.DMA,
        ],
    )
    def sc_kernel(z_hbm, ei_hbm, out_hbm, zpk_hbm,
                  pin0, pin1, pout0, pout1,
                  sem_pi0, sem_pi1, sem_po0, sem_po1,
                  idx_s0, idx_d0, idx_s1, idx_d1,
                  rows_s0, rows_d0, rows_s1, rows_d1,
                  out_v0, out_v1,
                  idx_ts, idx_td, rows_ts, rows_td, out_t,
                  sem_i0, sem_i1, sem_r0, sem_r1, sem_o0, sem_o1):
        cid = lax.axis_index("c")
        sid = lax.axis_index("s")
        wid = sid * NUM_CORES + cid
        tile_base = wid * E_PER_TILE
        idx_bufs = [(idx_s0, idx_d0, sem_i0), (idx_s1, idx_d1, sem_i1)]
        row_bufs = [(rows_s0, rows_d0, sem_r0), (rows_s1, rows_d1, sem_r1)]
        out_bufs = [(out_v0, sem_o0), (out_v1, sem_o1)]
        lane = lax.broadcasted_iota(jnp.int32, (L,), 0)

        def base_of(w):
            return pl.multiple_of(tile_base + w * W, L)

        def stage_idx(w, b):
            idx_s, idx_d, sem = idx_bufs[b]
            base = base_of(w)
            pltpu.async_copy(ei_hbm.at[0, pl.ds(base, W)], idx_s, sem)
            pltpu.async_copy(ei_hbm.at[1, pl.ds(base, W)], idx_d, sem)

        def wait_idx(b):
            idx_s, idx_d, sem = idx_bufs[b]
            pltpu.make_async_copy(ei_hbm.at[0, pl.ds(0, W)], idx_s, sem).wait()
            pltpu.make_async_copy(ei_hbm.at[1, pl.ds(0, W)], idx_d, sem).wait()

        def stage_rows(b):
            idx_s, idx_d, _ = idx_bufs[b]
            rows_s, rows_d, sem = row_bufs[b]
            wait_idx(b)
            pltpu.async_copy(zpk_hbm.at[idx_s], rows_s, sem)
            pltpu.async_copy(zpk_hbm.at[idx_d], rows_d, sem)

        def wait_rows(b):
            idx_s, idx_d, _ = idx_bufs[b]
            rows_s, rows_d, sem = row_bufs[b]
            pltpu.make_async_copy(zpk_hbm.at[idx_s], rows_s, sem).wait()
            pltpu.make_async_copy(zpk_hbm.at[idx_d], rows_d, sem).wait()

        def wait_out(b):
            out_v, sem = out_bufs[b]
            pltpu.make_async_copy(
                out_v, out_hbm.at[pl.ds(0, W)], sem).wait()

        def distance_group(rows_s, rows_d, out_v, e0):

            @plsc.parallel_loop(0, L, 1, unroll=4,
                                carry=jnp.zeros((L,), jnp.float32))
            def d2(k, res):
                e = e0 + k
                acc = jnp.zeros((L,), jnp.float32)
                for j in range(D_PK // L):
                    vs = plsc.bitcast(rows_s[e, pl.ds(j * L, L)],
                                      jnp.bfloat16)
                    vd = plsc.bitcast(rows_d[e, pl.ds(j * L, L)],
                                      jnp.bfloat16)
                    df = vs - vd
                    lo, hi = plsc.unpack(
                        df, format=plsc.PackFormat.INTERLEAVED)
                    acc = acc + lo * lo
                    acc = acc + hi * hi
                return jnp.where(lane == k, jnp.sum(acc), res)

            out_v[pl.ds(e0, L)] = jnp.exp(-_sqrt_vec(d2))

        def compute(w, b, drain_prev):
            rows_s, rows_d, _ = row_bufs[b]
            out_v, sem = out_bufs[b]
            if drain_prev:
                @pl.when(w >= 2)
                def _drain():
                    wait_out(b)

            @pl.loop(0, W // L)
            def _group(g):
                distance_group(rows_s, rows_d, out_v,
                               pl.multiple_of(g * L, L))

            pltpu.async_copy(out_v, out_hbm.at[pl.ds(base_of(w), W)], sem)

        stage_idx(0, 0)
        stage_idx(1, 1)

        r0 = sid * R_PER_TILE
        pins = [(pin0, sem_pi0), (pin1, sem_pi1)]
        pouts = [(pout0, sem_po0), (pout1, sem_po1)]
        n_chunks = R_PER_TILE // R_CHUNK

        def chunk_in(ci, b):
            pin, sem = pins[b]
            pltpu.async_copy(
                z_hbm.at[pl.ds(r0 + ci * R_CHUNK, R_CHUNK)], pin, sem)

        def chunk_in_wait(b):
            pin, sem = pins[b]
            pltpu.make_async_copy(
                z_hbm.at[pl.ds(0, R_CHUNK)], pin, sem).wait()

        def chunk_out_wait(b):
            pout, sem = pouts[b]
            pltpu.make_async_copy(
                pout, zpk_hbm.at[pl.ds(0, R_CHUNK)], sem).wait()

        chunk_in(0, 0)
        for ci in range(n_chunks):
            b = ci % 2
            if ci + 1 < n_chunks:
                chunk_in(ci + 1, 1 - b)
            chunk_in_wait(b)
            if ci >= 2:
                chunk_out_wait(b)
            pin, _ = pins[b]
            pout, sem_po = pouts[b]

            @plsc.parallel_loop(0, R_CHUNK, 1, unroll=4)
            def _row(r):
                for j in range(D_PK // L):
                    a = pin[r, pl.ds(j * 2 * L, L)]
                    bb = pin[r, pl.ds(j * 2 * L + L, L)]
                    pk = plsc.pack(a, bb, format=plsc.PackFormat.INTERLEAVED)
                    pout[r, pl.ds(j * L, L)] = plsc.bitcast(pk, jnp.int32)

            pltpu.async_copy(
                pout, zpk_hbm.at[pl.ds(r0 + ci * R_CHUNK, R_CHUNK)], sem_po)

        chunk_out_wait((n_chunks - 2) % 2)
        chunk_out_wait((n_chunks - 1) % 2)
        plsc.subcore_barrier()

        stage_rows(0)

        @pl.loop(0, (NWIN - 2) // 2)
        def _pair(p):
            w = p * 2
            stage_rows(1)
            wait_rows(0)
            stage_idx(w + 2, 0)
            compute(w, 0, drain_prev=True)
            wait_rows(1)
            stage_idx(w + 3, 1)
            stage_rows(0)
            compute(w + 1, 1, drain_prev=True)

        stage_rows(1)
        wait_rows(0)
        compute(NWIN - 2, 0, drain_prev=True)
        wait_rows(1)
        compute(NWIN - 1, 1, drain_prev=True)

        base_t = pl.multiple_of(tile_base + NWIN * W, L)
        pltpu.async_copy(ei_hbm.at[0, pl.ds(base_t, TAIL)], idx_ts, sem_i0)
        pltpu.async_copy(ei_hbm.at[1, pl.ds(base_t, TAIL)], idx_td, sem_i0)
        pltpu.make_async_copy(ei_hbm.at[0, pl.ds(0, TAIL)], idx_ts,
                              sem_i0).wait()
        pltpu.make_async_copy(ei_hbm.at[1, pl.ds(0, TAIL)], idx_td,
                              sem_i0).wait()
        pltpu.async_copy(zpk_hbm.at[idx_ts], rows_ts, sem_r0)
        pltpu.async_copy(zpk_hbm.at[idx_td], rows_td, sem_r0)
        pltpu.make_async_copy(zpk_hbm.at[idx_ts], rows_ts, sem_r0).wait()
        pltpu.make_async_copy(zpk_hbm.at[idx_td], rows_td, sem_r0).wait()
        distance_group(rows_ts, rows_td, out_t, 0)
        pltpu.sync_copy(out_t, out_hbm.at[pl.ds(base_t, TAIL)])

        wait_out(0)
        wait_out(1)

    probs, _ = sc_kernel(z, edge_index)
    return probs
